# Initial kernel scaffold; baseline (speedup 1.0000x reference)
import jax
import jax.numpy as jnp
from jax.experimental import pallas as pl


def kernel(edge_index, node_feature, rbf_tensor, dist, cutoff, W1, b1, Wf1, bf1, Wf2, bf2, W2, b2, W3, b3):
    raise NotImplementedError("write your pallas kernel here")


# speedup vs baseline: 1.0523x; 1.0523x over previous
import functools

import jax
import jax.numpy as jnp
from jax import lax
from jax.experimental import pallas as pl
from jax.experimental.pallas import tpu as pltpu
from jax.experimental.pallas import tpu_sc as plsc

_LOG2 = 0.6931471805599453


def _ssp(x):
    return jnp.log(jnp.exp(x) + 1.0) - _LOG2


def _h_body(x_ref, w_ref, b_ref, o_ref):
    o_ref[...] = (
        jnp.dot(x_ref[...], w_ref[...], preferred_element_type=jnp.float32)
        + b_ref[...]
    )


def _filter_body(rbf_ref, dsc_ref, wf1_ref, bf1_ref, wf2_ref, bf2_ref, o_ref):
    u = (
        jnp.dot(rbf_ref[...], wf1_ref[...], preferred_element_type=jnp.float32)
        + bf1_ref[...]
    )
    u = _ssp(u)
    v = (
        jnp.dot(u, wf2_ref[...], preferred_element_type=jnp.float32)
        + bf2_ref[...]
    )
    v = _ssp(v)
    o_ref[...] = v * (1.0 + jnp.cos(dsc_ref[...]))


def _final_body(p_ref, x0_ref, w2_ref, b2_ref, w3_ref, b3_ref, o_ref):
    n = x0_ref.shape[0]
    agg = p_ref[:n, :] + p_ref[n:, :]
    u = (
        jnp.dot(agg, w2_ref[...], preferred_element_type=jnp.float32)
        + b2_ref[...]
    )
    u = _ssp(u)
    o_ref[...] = (
        jnp.dot(u, w3_ref[...], preferred_element_type=jnp.float32)
        + b3_ref[...]
        + x0_ref[...]
    )


_NC = 2
_NS = 16
_NW = _NC * _NS
_CH = 128


def _make_sc_scatter(N, F, E):
    assert E % _CH == 0
    n_chunks = E // _CH
    nbase = n_chunks // _NW
    nrem = n_chunks % _NW
    rpt = (N // _NS) // 8 * 8
    rrem = N - rpt * _NS
    assert rrem % 8 == 0

    mesh = plsc.VectorSubcoreMesh(
        core_axis_name="c", subcore_axis_name="s", num_cores=_NC,
        num_subcores=_NS)

    @functools.partial(
        pl.kernel,
        out_type=jax.ShapeDtypeStruct((_NC * N, F), jnp.float32),
        mesh=mesh,
        scratch_types=[
            pltpu.VMEM((_CH,), jnp.int32),
            pltpu.VMEM((_CH,), jnp.int32),
            pltpu.VMEM((_CH, F), jnp.float32),
            pltpu.VMEM((_CH, F), jnp.float32),
            pltpu.VMEM_SHARED((N, F), jnp.float32),
            pltpu.SemaphoreType.DMA,
        ],
        compiler_params=pltpu.CompilerParams(use_tc_tiling_on_sc=False),
    )
    def sc_scatter(src_hbm, dst_hbm, h_hbm, w_hbm, zero_hbm, out_hbm,
                   srcv, dstv, hrows, wrows, acc, sem):
        cid = lax.axis_index("c")
        sid = lax.axis_index("s")
        wid = sid * _NC + cid

        row0 = sid * rpt
        pltpu.sync_copy(zero_hbm.at[pl.ds(row0, rpt)],
                        acc.at[pl.ds(row0, rpt)])
        if rrem:
            @pl.when(sid == 0)
            def _():
                pltpu.sync_copy(zero_hbm.at[pl.ds(_NS * rpt, rrem)],
                                acc.at[pl.ds(_NS * rpt, rrem)])
        plsc.subcore_barrier()

        cnt = nbase + jnp.where(wid < nrem, 1, 0)
        start = wid * nbase + jnp.minimum(wid, nrem)

        def chunk_body(i, carry):
            base = (start + i) * _CH
            pltpu.sync_copy(src_hbm.at[pl.ds(base, _CH)], srcv)
            pltpu.sync_copy(dst_hbm.at[pl.ds(base, _CH)], dstv)
            pltpu.async_copy(h_hbm.at[srcv], hrows, sem).wait()
            pltpu.sync_copy(w_hbm.at[pl.ds(base, _CH)], wrows)

            def mul_row(r, c2):
                for k in range(F // 16):
                    sl = pl.ds(k * 16, 16)
                    hrows[r, sl] = hrows[r, sl] * wrows[r, sl]
                return c2

            lax.fori_loop(0, _CH, mul_row, 0)
            pltpu.sync_copy(hrows, acc.at[dstv], add=True)
            return carry

        lax.fori_loop(0, cnt, chunk_body, 0)
        plsc.subcore_barrier()
        pltpu.sync_copy(
            acc.at[pl.ds(row0, rpt)],
            out_hbm.at[pl.ds(cid * N + row0, rpt)])
        if rrem:
            @pl.when(sid == 0)
            def _():
                pltpu.sync_copy(
                    acc.at[pl.ds(_NS * rpt, rrem)],
                    out_hbm.at[pl.ds(cid * N + _NS * rpt, rrem)])

    return sc_scatter


def kernel(edge_index, node_feature, rbf_tensor, dist, cutoff,
           W1, b1, Wf1, bf1, Wf2, bf2, W2, b2, W3, b3):
    N, F = node_feature.shape
    E = edge_index.shape[1]
    R = rbf_tensor.shape[1]

    h = pl.pallas_call(
        _h_body,
        out_shape=jax.ShapeDtypeStruct((N, F), jnp.float32),
    )(node_feature, W1.T, b1.reshape(1, F))

    BE = 640
    assert E % BE == 0
    dsc = dist * (jnp.float32(3.14159265) / jnp.asarray(cutoff, jnp.float32))
    w = pl.pallas_call(
        _filter_body,
        grid=(E // BE,),
        in_specs=[
            pl.BlockSpec((BE, R), lambda i: (i, 0)),
            pl.BlockSpec((BE, 1), lambda i: (i, 0)),
            pl.BlockSpec((R, F), lambda i: (0, 0)),
            pl.BlockSpec((1, F), lambda i: (0, 0)),
            pl.BlockSpec((F, F), lambda i: (0, 0)),
            pl.BlockSpec((1, F), lambda i: (0, 0)),
        ],
        out_specs=pl.BlockSpec((BE, F), lambda i: (i, 0)),
        out_shape=jax.ShapeDtypeStruct((E, F), jnp.float32),
    )(rbf_tensor, dsc, Wf1.T, bf1.reshape(1, F), Wf2.T, bf2.reshape(1, F))

    src = edge_index[1]
    dst = edge_index[0]
    zeros = jnp.zeros((N, F), jnp.float32)
    partial = _make_sc_scatter(N, F, E)(src, dst, h, w, zeros)

    out = pl.pallas_call(
        _final_body,
        out_shape=jax.ShapeDtypeStruct((N, F), jnp.float32),
    )(partial, node_feature, W2.T, b2.reshape(1, F), W3.T, b3.reshape(1, F))
    return out


# speedup vs baseline: 1.2571x; 1.1946x over previous
import functools

import jax
import jax.numpy as jnp
from jax import lax
from jax.experimental import pallas as pl
from jax.experimental.pallas import tpu as pltpu
from jax.experimental.pallas import tpu_sc as plsc

_LOG2 = 0.6931471805599453


def _ssp(x):
    return jnp.log(jnp.exp(x) + 1.0) - _LOG2


def _h_body(x_ref, w_ref, b_ref, o_ref):
    o_ref[...] = (
        jnp.dot(x_ref[...], w_ref[...], preferred_element_type=jnp.float32)
        + b_ref[...]
    )


def _env_body(dsc_ref, o_ref):
    o_ref[...] = 1.0 + jnp.cos(dsc_ref[...])


def _filter_body(rbf_ref, env_ref, wf1_ref, bf1_ref, wf2_ref, bf2_ref, o_ref):
    u = (
        jnp.dot(rbf_ref[...], wf1_ref[...], preferred_element_type=jnp.float32)
        + bf1_ref[...]
    )
    u = _ssp(u)
    v = (
        jnp.dot(u, wf2_ref[...], preferred_element_type=jnp.float32)
        + bf2_ref[...]
    )
    v = _ssp(v)
    o_ref[...] = v * env_ref[...]


def _final_body(p_ref, x0_ref, w2_ref, b2_ref, w3_ref, b3_ref, o_ref):
    n = x0_ref.shape[0]
    agg = p_ref[:n, :] + p_ref[n:, :]
    u = (
        jnp.dot(agg, w2_ref[...], preferred_element_type=jnp.float32)
        + b2_ref[...]
    )
    u = _ssp(u)
    o_ref[...] = (
        jnp.dot(u, w3_ref[...], preferred_element_type=jnp.float32)
        + b3_ref[...]
        + x0_ref[...]
    )


_NC = 2
_NS = 16
_NW = _NC * _NS
_CH = 128


def _make_sc_scatter(N, F, E):
    assert E % _CH == 0
    n_chunks = E // _CH
    nbase = n_chunks // _NW
    nrem = n_chunks % _NW
    rpt = (N // _NS) // 8 * 8
    rrem = N - rpt * _NS
    assert rrem % 8 == 0

    mesh = plsc.VectorSubcoreMesh(
        core_axis_name="c", subcore_axis_name="s", num_cores=_NC,
        num_subcores=_NS)

    @functools.partial(
        pl.kernel,
        out_type=jax.ShapeDtypeStruct((_NC * N, F), jnp.float32),
        mesh=mesh,
        scratch_types=[
            pltpu.VMEM((_CH,), jnp.int32),
            pltpu.VMEM((_CH,), jnp.int32),
            pltpu.VMEM((_CH, F), jnp.float32),
            pltpu.VMEM((_CH, F), jnp.float32),
            pltpu.VMEM_SHARED((N, F), jnp.float32),
            pltpu.SemaphoreType.DMA,
        ],
        compiler_params=pltpu.CompilerParams(use_tc_tiling_on_sc=False),
    )
    def sc_scatter(src_hbm, dst_hbm, h_hbm, w_hbm, zero_hbm, out_hbm,
                   srcv, dstv, hrows, wrows, acc, sem):
        cid = lax.axis_index("c")
        sid = lax.axis_index("s")
        wid = sid * _NC + cid

        row0 = sid * rpt
        pltpu.sync_copy(zero_hbm.at[pl.ds(row0, rpt)],
                        acc.at[pl.ds(row0, rpt)])
        if rrem:
            @pl.when(sid == 0)
            def _():
                pltpu.sync_copy(zero_hbm.at[pl.ds(_NS * rpt, rrem)],
                                acc.at[pl.ds(_NS * rpt, rrem)])
        plsc.subcore_barrier()

        cnt = nbase + jnp.where(wid < nrem, 1, 0)
        start = wid * nbase + jnp.minimum(wid, nrem)

        def chunk_body(i, carry):
            base = (start + i) * _CH
            pltpu.sync_copy(src_hbm.at[pl.ds(base, _CH)], srcv)
            pltpu.sync_copy(dst_hbm.at[pl.ds(base, _CH)], dstv)
            pltpu.async_copy(h_hbm.at[srcv], hrows, sem).wait()
            pltpu.sync_copy(w_hbm.at[pl.ds(base, _CH)], wrows)

            def mul_row(r, c2):
                for k in range(F // 16):
                    sl = pl.ds(k * 16, 16)
                    hrows[r, sl] = hrows[r, sl] * wrows[r, sl]
                return c2

            lax.fori_loop(0, _CH, mul_row, 0)
            pltpu.sync_copy(hrows, acc.at[dstv], add=True)
            return carry

        lax.fori_loop(0, cnt, chunk_body, 0)
        plsc.subcore_barrier()
        pltpu.sync_copy(
            acc.at[pl.ds(row0, rpt)],
            out_hbm.at[pl.ds(cid * N + row0, rpt)])
        if rrem:
            @pl.when(sid == 0)
            def _():
                pltpu.sync_copy(
                    acc.at[pl.ds(_NS * rpt, rrem)],
                    out_hbm.at[pl.ds(cid * N + _NS * rpt, rrem)])

    return sc_scatter


def kernel(edge_index, node_feature, rbf_tensor, dist, cutoff,
           W1, b1, Wf1, bf1, Wf2, bf2, W2, b2, W3, b3):
    N, F = node_feature.shape
    E = edge_index.shape[1]
    R = rbf_tensor.shape[1]

    h = pl.pallas_call(
        _h_body,
        out_shape=jax.ShapeDtypeStruct((N, F), jnp.float32),
    )(node_feature, W1.T, b1.reshape(1, F))

    BE = 640
    assert E % BE == 0
    dsc = dist * (jnp.float32(3.14159265) / jnp.asarray(cutoff, jnp.float32))
    env = pl.pallas_call(
        _env_body,
        out_shape=jax.ShapeDtypeStruct((E // 128, 128), jnp.float32),
    )(dsc.reshape(E // 128, 128)).reshape(E, 1)
    w = pl.pallas_call(
        _filter_body,
        grid=(E // BE,),
        in_specs=[
            pl.BlockSpec((BE, R), lambda i: (i, 0)),
            pl.BlockSpec((BE, 1), lambda i: (i, 0)),
            pl.BlockSpec((R, F), lambda i: (0, 0)),
            pl.BlockSpec((1, F), lambda i: (0, 0)),
            pl.BlockSpec((F, F), lambda i: (0, 0)),
            pl.BlockSpec((1, F), lambda i: (0, 0)),
        ],
        out_specs=pl.BlockSpec((BE, F), lambda i: (i, 0)),
        out_shape=jax.ShapeDtypeStruct((E, F), jnp.float32),
    )(rbf_tensor, env, Wf1.T, bf1.reshape(1, F), Wf2.T, bf2.reshape(1, F))

    src = edge_index[1]
    dst = edge_index[0]
    zeros = jnp.zeros((N, F), jnp.float32)
    partial = _make_sc_scatter(N, F, E)(src, dst, h, w, zeros)

    out = pl.pallas_call(
        _final_body,
        out_shape=jax.ShapeDtypeStruct((N, F), jnp.float32),
    )(partial, node_feature, W2.T, b2.reshape(1, F), W3.T, b3.reshape(1, F))
    return out


# speedup vs baseline: 1.7887x; 1.4229x over previous
import functools

import jax
import jax.numpy as jnp
from jax import lax
from jax.experimental import pallas as pl
from jax.experimental.pallas import tpu as pltpu
from jax.experimental.pallas import tpu_sc as plsc

_LOG2 = 0.6931471805599453


def _ssp(x):
    return jnp.log(jnp.exp(x) + 1.0) - _LOG2


def _h_body(x_ref, w_ref, b_ref, o_ref):
    o_ref[...] = (
        jnp.dot(x_ref[...], w_ref[...], preferred_element_type=jnp.float32)
        + b_ref[...]
    )


def _env_body(dsc_ref, o_ref):
    o_ref[...] = 1.0 + jnp.cos(dsc_ref[...])


def _filter_body(rbft_ref, env_ref, wf1_ref, bf1_ref, wf2_ref, bf2_ref,
                 o_ref):
    u = lax.dot_general(
        rbft_ref[...], wf1_ref[...], (((0,), (0,)), ((), ())),
        preferred_element_type=jnp.float32,
    ) + bf1_ref[...]
    u = _ssp(u)
    v = (
        jnp.dot(u, wf2_ref[...], preferred_element_type=jnp.float32)
        + bf2_ref[...]
    )
    v = _ssp(v)
    be, f = v.shape
    v3 = jnp.reshape(v, (be // 128, 128, f))
    e3 = lax.broadcast_in_dim(env_ref[0], (be // 128, 128, f), (0, 1))
    o_ref[:, 0:64] = jnp.reshape(v3 * e3, (be, f))


def _final_body(p_ref, x0_ref, w2_ref, b2_ref, w3_ref, b3_ref, o_ref):
    n = x0_ref.shape[0]
    agg = p_ref[:n, :] + p_ref[n:, :]
    u = (
        jnp.dot(agg, w2_ref[...], preferred_element_type=jnp.float32)
        + b2_ref[...]
    )
    u = _ssp(u)
    o_ref[...] = (
        jnp.dot(u, w3_ref[...], preferred_element_type=jnp.float32)
        + b3_ref[...]
        + x0_ref[...]
    )


_NC = 2
_NS = 16
_NW = _NC * _NS
_CH = 128


def _make_sc_scatter(N, F, E):
    assert E % _CH == 0
    n_chunks = E // _CH
    nbase = n_chunks // _NW
    nrem = n_chunks % _NW
    rpt = (N // _NS) // 8 * 8
    rrem = N - rpt * _NS
    assert rrem % 8 == 0

    mesh = plsc.VectorSubcoreMesh(
        core_axis_name="c", subcore_axis_name="s", num_cores=_NC,
        num_subcores=_NS)

    @functools.partial(
        pl.kernel,
        out_type=jax.ShapeDtypeStruct((_NC * N, F), jnp.float32),
        mesh=mesh,
        scratch_types=[
            pltpu.VMEM((_CH,), jnp.int32),
            pltpu.VMEM((_CH,), jnp.int32),
            pltpu.VMEM((_CH, F), jnp.float32),
            pltpu.VMEM((_CH, 128), jnp.float32),
            pltpu.VMEM_SHARED((N, F), jnp.float32),
            pltpu.SemaphoreType.DMA,
        ],
        compiler_params=pltpu.CompilerParams(use_tc_tiling_on_sc=False),
    )
    def sc_scatter(src_hbm, dst_hbm, h_hbm, w_hbm, zero_hbm,
                   out_hbm, srcv, dstv, hrows, wrows, acc, sem):
        cid = lax.axis_index("c")
        sid = lax.axis_index("s")
        wid = sid * _NC + cid

        row0 = sid * rpt
        pltpu.sync_copy(zero_hbm.at[pl.ds(row0, rpt)],
                        acc.at[pl.ds(row0, rpt)])
        if rrem:
            @pl.when(sid == 0)
            def _():
                pltpu.sync_copy(zero_hbm.at[pl.ds(_NS * rpt, rrem)],
                                acc.at[pl.ds(_NS * rpt, rrem)])
        plsc.subcore_barrier()

        cnt = nbase + jnp.where(wid < nrem, 1, 0)
        start = wid * nbase + jnp.minimum(wid, nrem)

        def chunk_body(i, carry):
            base = (start + i) * _CH
            pltpu.sync_copy(src_hbm.at[pl.ds(base, _CH)], srcv)
            pltpu.sync_copy(dst_hbm.at[pl.ds(base, _CH)], dstv)
            pltpu.async_copy(h_hbm.at[srcv], hrows, sem).wait()
            pltpu.sync_copy(w_hbm.at[pl.ds(base, _CH)], wrows)

            def mul_row(r, c2):
                for k in range(F // 16):
                    sl = pl.ds(k * 16, 16)
                    hrows[r, sl] = hrows[r, sl] * wrows[r, sl]
                return c2

            lax.fori_loop(0, _CH, mul_row, 0)
            pltpu.sync_copy(hrows, acc.at[dstv], add=True)
            return carry

        lax.fori_loop(0, cnt, chunk_body, 0)
        plsc.subcore_barrier()
        pltpu.sync_copy(
            acc.at[pl.ds(row0, rpt)],
            out_hbm.at[pl.ds(cid * N + row0, rpt)])
        if rrem:
            @pl.when(sid == 0)
            def _():
                pltpu.sync_copy(
                    acc.at[pl.ds(_NS * rpt, rrem)],
                    out_hbm.at[pl.ds(cid * N + _NS * rpt, rrem)])

    return sc_scatter


def kernel(edge_index, node_feature, rbf_tensor, dist, cutoff,
           W1, b1, Wf1, bf1, Wf2, bf2, W2, b2, W3, b3):
    N, F = node_feature.shape
    E = edge_index.shape[1]
    R = rbf_tensor.shape[1]

    h = pl.pallas_call(
        _h_body,
        out_shape=jax.ShapeDtypeStruct((N, F), jnp.float32),
    )(node_feature, W1.T, b1.reshape(1, F))

    dsc = dist.reshape(E // 128, 128) * (
        jnp.float32(3.14159265) / jnp.asarray(cutoff, jnp.float32))
    env2d = pl.pallas_call(
        _env_body,
        out_shape=jax.ShapeDtypeStruct((E // 128, 128), jnp.float32),
    )(dsc)

    BE = 640
    assert E % BE == 0 and BE % 128 == 0
    w = pl.pallas_call(
        _filter_body,
        grid=(E // BE,),
        in_specs=[
            pl.BlockSpec((R, BE), lambda i: (0, i)),
            pl.BlockSpec((1, BE // 128, 128), lambda i: (i, 0, 0)),
            pl.BlockSpec((R, F), lambda i: (0, 0)),
            pl.BlockSpec((1, F), lambda i: (0, 0)),
            pl.BlockSpec((F, F), lambda i: (0, 0)),
            pl.BlockSpec((1, F), lambda i: (0, 0)),
        ],
        out_specs=pl.BlockSpec((BE, 128), lambda i: (i, 0)),
        out_shape=jax.ShapeDtypeStruct((E, 128), jnp.float32),
    )(rbf_tensor.T, env2d.reshape(E // BE, BE // 128, 128), Wf1.T,
      bf1.reshape(1, F), Wf2.T, bf2.reshape(1, F))

    src = edge_index[1]
    dst = edge_index[0]
    zeros = jnp.zeros((N, F), jnp.float32)
    partial = _make_sc_scatter(N, F, E)(src, dst, h, w, zeros)

    out = pl.pallas_call(
        _final_body,
        out_shape=jax.ShapeDtypeStruct((N, F), jnp.float32),
    )(partial, node_feature, W2.T, b2.reshape(1, F), W3.T, b3.reshape(1, F))
    return out


# speedup vs baseline: 2.0466x; 1.1442x over previous
import functools

import jax
import jax.numpy as jnp
from jax import lax
from jax.experimental import pallas as pl
from jax.experimental.pallas import tpu as pltpu
from jax.experimental.pallas import tpu_sc as plsc

_LOG2 = 0.6931471805599453


def _ssp(x):
    return jnp.log(jnp.exp(x) + 1.0) - _LOG2


def _h_body(x_ref, w_ref, b_ref, o_ref):
    o_ref[...] = (
        jnp.dot(x_ref[...], w_ref[...], preferred_element_type=jnp.float32)
        + b_ref[...]
    )


def _env_body(dsc_ref, o_ref):
    o_ref[...] = 1.0 + jnp.cos(dsc_ref[...])


def _filter_body(rbft_ref, env_ref, wf1_ref, bf1_ref, wf2_ref, bf2_ref,
                 o_ref):
    u = lax.dot_general(
        rbft_ref[...], wf1_ref[...], (((0,), (0,)), ((), ())),
        preferred_element_type=jnp.float32,
    ) + bf1_ref[...]
    u = _ssp(u)
    v = (
        jnp.dot(u, wf2_ref[...], preferred_element_type=jnp.float32)
        + bf2_ref[...]
    )
    v = _ssp(v)
    be, f = v.shape
    v3 = jnp.reshape(v, (be // 128, 128, f))
    e3 = lax.broadcast_in_dim(env_ref[0], (be // 128, 128, f), (0, 1))
    o_ref[:, 0:64] = jnp.reshape(v3 * e3, (be, f))


def _final_body(p_ref, x0_ref, w2_ref, b2_ref, w3_ref, b3_ref, o_ref):
    n = x0_ref.shape[0]
    agg = p_ref[:n, :] + p_ref[n:, :]
    u = (
        jnp.dot(agg, w2_ref[...], preferred_element_type=jnp.float32)
        + b2_ref[...]
    )
    u = _ssp(u)
    o_ref[...] = (
        jnp.dot(u, w3_ref[...], preferred_element_type=jnp.float32)
        + b3_ref[...]
        + x0_ref[...]
    )


_NC = 2
_NS = 16
_NW = _NC * _NS
_CH = 128


def _make_sc_scatter(N, F, E):
    assert E % _CH == 0
    n_chunks = E // _CH
    T = (n_chunks + _NW - 1) // _NW
    rpt = (N // _NS) // 8 * 8
    rrem = N - rpt * _NS
    assert rrem % 8 == 0

    mesh = plsc.VectorSubcoreMesh(
        core_axis_name="c", subcore_axis_name="s", num_cores=_NC,
        num_subcores=_NS)

    @functools.partial(
        pl.kernel,
        out_type=jax.ShapeDtypeStruct((_NC * N, F), jnp.float32),
        mesh=mesh,
        scratch_types=[
            pltpu.VMEM((2, 2, _CH), jnp.int32),
            pltpu.VMEM((2, _CH, F), jnp.float32),
            pltpu.VMEM((2, _CH, 128), jnp.float32),
            pltpu.VMEM_SHARED((N, F), jnp.float32),
            pltpu.SemaphoreType.DMA((2,)),
            pltpu.SemaphoreType.DMA((2,)),
            pltpu.SemaphoreType.DMA((2,)),
        ],
        compiler_params=pltpu.CompilerParams(use_tc_tiling_on_sc=False),
    )
    def sc_scatter(ei_hbm, h_hbm, w_hbm, zero_hbm, out_hbm,
                   eib, hrows, wrows, acc, sidx, sw, sg):
        cid = lax.axis_index("c")
        sid = lax.axis_index("s")
        wid = sid * _NC + cid

        row0 = sid * rpt
        pltpu.sync_copy(zero_hbm.at[pl.ds(row0, rpt)],
                        acc.at[pl.ds(row0, rpt)])
        if rrem:
            @pl.when(sid == 0)
            def _():
                pltpu.sync_copy(zero_hbm.at[pl.ds(_NS * rpt, rrem)],
                                acc.at[pl.ds(_NS * rpt, rrem)])
        plsc.subcore_barrier()

        def slot_base(j):
            g = wid * T + j
            return jnp.minimum(g, n_chunks - 1) * _CH, g < n_chunks

        def start_pre(j, b):
            base, _ = slot_base(j)
            pltpu.async_copy(ei_hbm.at[:, pl.ds(base, _CH)], eib.at[b],
                             sidx.at[b])
            pltpu.async_copy(w_hbm.at[pl.ds(base, _CH)], wrows.at[b],
                             sw.at[b])

        def wait_pre(b):
            base0 = 0
            pltpu.make_async_copy(ei_hbm.at[:, pl.ds(base0, _CH)],
                                  eib.at[b], sidx.at[b]).wait()
            pltpu.make_async_copy(w_hbm.at[pl.ds(base0, _CH)],
                                  wrows.at[b], sw.at[b]).wait()

        def start_gather(b):
            pltpu.async_copy(h_hbm.at[eib.at[b, 0]], hrows.at[b], sg.at[b])

        def wait_gather(b):
            pltpu.make_async_copy(h_hbm.at[eib.at[b, 0]], hrows.at[b],
                                  sg.at[b]).wait()

        start_pre(0, 0)
        wait_pre(0)
        start_gather(0)
        start_pre(1, 1)

        for j in range(T):
            b = j % 2
            ob = 1 - b
            if j + 1 < T:
                wait_pre(ob)
                start_gather(ob)
            wait_gather(b)

            def mul_row(r, c2):
                for rr in range(2):
                    for k in range(F // 16):
                        sl = pl.ds(k * 16, 16)
                        hrows[b, 2 * r + rr, sl] = (
                            hrows[b, 2 * r + rr, sl]
                            * wrows[b, 2 * r + rr, sl])
                return c2

            lax.fori_loop(0, _CH // 2, mul_row, 0)
            _, live = slot_base(j)

            @pl.when(live)
            def _():
                pltpu.sync_copy(hrows.at[b], acc.at[eib.at[b, 1]], add=True)

            if j + 2 < T:
                start_pre(j + 2, b)

        plsc.subcore_barrier()
        pltpu.sync_copy(
            acc.at[pl.ds(row0, rpt)],
            out_hbm.at[pl.ds(cid * N + row0, rpt)])
        if rrem:
            @pl.when(sid == 0)
            def _():
                pltpu.sync_copy(
                    acc.at[pl.ds(_NS * rpt, rrem)],
                    out_hbm.at[pl.ds(cid * N + _NS * rpt, rrem)])

    return sc_scatter


def kernel(edge_index, node_feature, rbf_tensor, dist, cutoff,
           W1, b1, Wf1, bf1, Wf2, bf2, W2, b2, W3, b3):
    N, F = node_feature.shape
    E = edge_index.shape[1]
    R = rbf_tensor.shape[1]

    h = pl.pallas_call(
        _h_body,
        out_shape=jax.ShapeDtypeStruct((N, F), jnp.float32),
    )(node_feature, W1.T, b1.reshape(1, F))

    dsc = dist.reshape(E // 128, 128) * (
        jnp.float32(3.14159265) / jnp.asarray(cutoff, jnp.float32))
    env2d = pl.pallas_call(
        _env_body,
        out_shape=jax.ShapeDtypeStruct((E // 128, 128), jnp.float32),
    )(dsc)

    BE = 640
    assert E % BE == 0 and BE % 128 == 0
    w = pl.pallas_call(
        _filter_body,
        grid=(E // BE,),
        in_specs=[
            pl.BlockSpec((R, BE), lambda i: (0, i)),
            pl.BlockSpec((1, BE // 128, 128), lambda i: (i, 0, 0)),
            pl.BlockSpec((R, F), lambda i: (0, 0)),
            pl.BlockSpec((1, F), lambda i: (0, 0)),
            pl.BlockSpec((F, F), lambda i: (0, 0)),
            pl.BlockSpec((1, F), lambda i: (0, 0)),
        ],
        out_specs=pl.BlockSpec((BE, 128), lambda i: (i, 0)),
        out_shape=jax.ShapeDtypeStruct((E, 128), jnp.float32),
    )(rbf_tensor.T, env2d.reshape(E // BE, BE // 128, 128), Wf1.T,
      bf1.reshape(1, F), Wf2.T, bf2.reshape(1, F))

    ei = jnp.stack([edge_index[1], edge_index[0]])
    zeros = jnp.zeros((N, F), jnp.float32)
    partial = _make_sc_scatter(N, F, E)(ei, h, w, zeros)

    out = pl.pallas_call(
        _final_body,
        out_shape=jax.ShapeDtypeStruct((N, F), jnp.float32),
    )(partial, node_feature, W2.T, b2.reshape(1, F), W3.T, b3.reshape(1, F))
    return out


# speedup vs baseline: 2.3928x; 1.1692x over previous
import functools

import jax
import jax.numpy as jnp
from jax import lax
from jax.experimental import pallas as pl
from jax.experimental.pallas import tpu as pltpu
from jax.experimental.pallas import tpu_sc as plsc

_LOG2 = 0.6931471805599453


def _ssp(x):
    return jnp.log(jnp.exp(x) + 1.0) - _LOG2


def _h_body(x_ref, w_ref, b_ref, o_ref):
    o_ref[...] = (
        jnp.dot(x_ref[...], w_ref[...], preferred_element_type=jnp.float32)
        + b_ref[...]
    )


def _env_body(dsc_ref, o_ref):
    o_ref[...] = 1.0 + jnp.cos(dsc_ref[...])


def _filter_body(rbft_ref, env_ref, wf1_ref, bf1_ref, wf2_ref, bf2_ref,
                 o_ref):
    u = lax.dot_general(
        rbft_ref[...], wf1_ref[...], (((0,), (0,)), ((), ())),
        preferred_element_type=jnp.float32,
    ) + bf1_ref[...]
    u = _ssp(u)
    v = (
        jnp.dot(u, wf2_ref[...], preferred_element_type=jnp.float32)
        + bf2_ref[...]
    )
    v = _ssp(v)
    be, f = v.shape
    v3 = jnp.reshape(v, (be // 128, 128, f))
    e3 = lax.broadcast_in_dim(env_ref[0], (be // 128, 128, f), (0, 1))
    o_ref[:, 0:64] = jnp.reshape(v3 * e3, (be, f))


def _final_body(p0_ref, p1_ref, x0_ref, w2_ref, b2_ref, w3_ref, b3_ref,
                o_ref):
    n = x0_ref.shape[0]
    agg = (p0_ref[:n, :] + p0_ref[n:, :]) + (p1_ref[:n, :] + p1_ref[n:, :])
    u = (
        jnp.dot(agg, w2_ref[...], preferred_element_type=jnp.float32)
        + b2_ref[...]
    )
    u = _ssp(u)
    o_ref[...] = (
        jnp.dot(u, w3_ref[...], preferred_element_type=jnp.float32)
        + b3_ref[...]
        + x0_ref[...]
    )


_NC = 2
_NS = 16
_NW = _NC * _NS
_CH = 128


def _make_sc_scatter(N, F, off_chunks, cnt_chunks):
    T = (cnt_chunks + _NW - 1) // _NW
    rpt = (N // _NS) // 8 * 8
    rrem = N - rpt * _NS
    assert rrem % 8 == 0

    mesh = plsc.VectorSubcoreMesh(
        core_axis_name="c", subcore_axis_name="s", num_cores=_NC,
        num_subcores=_NS)

    @functools.partial(
        pl.kernel,
        out_type=jax.ShapeDtypeStruct((_NC * N, F), jnp.float32),
        mesh=mesh,
        scratch_types=[
            pltpu.VMEM((2, 2, _CH), jnp.int32),
            pltpu.VMEM((2, _CH, F), jnp.float32),
            pltpu.VMEM((2, _CH, 128), jnp.float32),
            pltpu.VMEM_SHARED((N, F), jnp.float32),
            pltpu.SemaphoreType.DMA((2,)),
            pltpu.SemaphoreType.DMA((2,)),
            pltpu.SemaphoreType.DMA((2,)),
        ],
        compiler_params=pltpu.CompilerParams(use_tc_tiling_on_sc=False),
    )
    def sc_scatter(ei_hbm, h_hbm, w_hbm, zero_hbm, out_hbm,
                   eib, hrows, wrows, acc, sidx, sw, sg):
        cid = lax.axis_index("c")
        sid = lax.axis_index("s")
        wid = sid * _NC + cid

        row0 = sid * rpt
        pltpu.sync_copy(zero_hbm.at[pl.ds(row0, rpt)],
                        acc.at[pl.ds(row0, rpt)])
        if rrem:
            @pl.when(sid == 0)
            def _():
                pltpu.sync_copy(zero_hbm.at[pl.ds(_NS * rpt, rrem)],
                                acc.at[pl.ds(_NS * rpt, rrem)])
        plsc.subcore_barrier()

        def slot_base(j):
            g = wid * T + j
            gc = jnp.minimum(g, cnt_chunks - 1)
            return (off_chunks + gc) * _CH, gc * _CH, g < cnt_chunks

        def start_pre(j, b):
            ebase, wbase, _ = slot_base(j)
            pltpu.async_copy(ei_hbm.at[:, pl.ds(ebase, _CH)], eib.at[b],
                             sidx.at[b])
            pltpu.async_copy(w_hbm.at[pl.ds(wbase, _CH)], wrows.at[b],
                             sw.at[b])

        def wait_pre(b):
            base0 = 0
            pltpu.make_async_copy(ei_hbm.at[:, pl.ds(base0, _CH)],
                                  eib.at[b], sidx.at[b]).wait()
            pltpu.make_async_copy(w_hbm.at[pl.ds(base0, _CH)],
                                  wrows.at[b], sw.at[b]).wait()

        def start_gather(b):
            pltpu.async_copy(h_hbm.at[eib.at[b, 0]], hrows.at[b], sg.at[b])

        def wait_gather(b):
            pltpu.make_async_copy(h_hbm.at[eib.at[b, 0]], hrows.at[b],
                                  sg.at[b]).wait()

        start_pre(0, 0)
        wait_pre(0)
        start_gather(0)
        start_pre(1, 1)

        for j in range(T):
            b = j % 2
            ob = 1 - b
            if j + 1 < T:
                wait_pre(ob)
                start_gather(ob)
            wait_gather(b)

            def mul_row(r, c2):
                for rr in range(2):
                    for k in range(F // 16):
                        sl = pl.ds(k * 16, 16)
                        hrows[b, 2 * r + rr, sl] = (
                            hrows[b, 2 * r + rr, sl]
                            * wrows[b, 2 * r + rr, sl])
                return c2

            lax.fori_loop(0, _CH // 2, mul_row, 0)
            _, _, live = slot_base(j)

            @pl.when(live)
            def _():
                pltpu.sync_copy(hrows.at[b], acc.at[eib.at[b, 1]], add=True)

            if j + 2 < T:
                start_pre(j + 2, b)

        plsc.subcore_barrier()
        pltpu.sync_copy(
            acc.at[pl.ds(row0, rpt)],
            out_hbm.at[pl.ds(cid * N + row0, rpt)])
        if rrem:
            @pl.when(sid == 0)
            def _():
                pltpu.sync_copy(
                    acc.at[pl.ds(_NS * rpt, rrem)],
                    out_hbm.at[pl.ds(cid * N + _NS * rpt, rrem)])

    return sc_scatter


def kernel(edge_index, node_feature, rbf_tensor, dist, cutoff,
           W1, b1, Wf1, bf1, Wf2, bf2, W2, b2, W3, b3):
    N, F = node_feature.shape
    E = edge_index.shape[1]
    R = rbf_tensor.shape[1]

    h = pl.pallas_call(
        _h_body,
        out_shape=jax.ShapeDtypeStruct((N, F), jnp.float32),
    )(node_feature, W1.T, b1.reshape(1, F))

    dsc = dist.reshape(E // 128, 128) * (
        jnp.float32(3.14159265) / jnp.asarray(cutoff, jnp.float32))
    env2d = pl.pallas_call(
        _env_body,
        out_shape=jax.ShapeDtypeStruct((E // 128, 128), jnp.float32),
    )(dsc)

    BE = 640
    assert E % (2 * BE) == 0 and BE % 128 == 0
    env3d = env2d.reshape(E // BE, BE // 128, 128)
    rbft = rbf_tensor.T

    def filter_half(goff):
        return pl.pallas_call(
            _filter_body,
            grid=(E // BE // 2,),
            in_specs=[
                pl.BlockSpec((R, BE), lambda i: (0, i + goff)),
                pl.BlockSpec((1, BE // 128, 128),
                             lambda i: (i + goff, 0, 0)),
                pl.BlockSpec((R, F), lambda i: (0, 0)),
                pl.BlockSpec((1, F), lambda i: (0, 0)),
                pl.BlockSpec((F, F), lambda i: (0, 0)),
                pl.BlockSpec((1, F), lambda i: (0, 0)),
            ],
            out_specs=pl.BlockSpec((BE, 128), lambda i: (i, 0)),
            out_shape=jax.ShapeDtypeStruct((E // 2, 128), jnp.float32),
        )(rbft, env3d, Wf1.T, bf1.reshape(1, F), Wf2.T, bf2.reshape(1, F))

    w0 = filter_half(0)
    w1 = filter_half(E // BE // 2)

    ei = jnp.stack([edge_index[1], edge_index[0]])
    zeros = jnp.zeros((N, F), jnp.float32)
    half_chunks = E // _CH // 2
    p0 = _make_sc_scatter(N, F, 0, half_chunks)(ei, h, w0, zeros)
    p1 = _make_sc_scatter(N, F, half_chunks, half_chunks)(ei, h, w1, zeros)

    out = pl.pallas_call(
        _final_body,
        out_shape=jax.ShapeDtypeStruct((N, F), jnp.float32),
    )(p0, p1, node_feature, W2.T, b2.reshape(1, F), W3.T,
      b3.reshape(1, F))
    return out


# speedup vs baseline: 2.5713x; 1.0746x over previous
import functools

import jax
import jax.numpy as jnp
from jax import lax
from jax.experimental import pallas as pl
from jax.experimental.pallas import tpu as pltpu
from jax.experimental.pallas import tpu_sc as plsc

_LOG2 = 0.6931471805599453


def _ssp(x):
    return jnp.log(jnp.exp(x) + 1.0) - _LOG2


def _h_body(x_ref, w_ref, b_ref, o_ref):
    o_ref[...] = (
        jnp.dot(x_ref[...], w_ref[...], preferred_element_type=jnp.float32)
        + b_ref[...]
    )


def _env_body(s_ref, d_ref, o_ref):
    o_ref[...] = 1.0 + jnp.cos(d_ref[...] * s_ref[0, 0])


def _filter_body(rbft_ref, env_ref, wf1_ref, bf1_ref, wf2_ref, bf2_ref,
                 o_ref):
    u = lax.dot_general(
        rbft_ref[...], wf1_ref[...], (((0,), (0,)), ((), ())),
        preferred_element_type=jnp.float32,
    ) + bf1_ref[...]
    u = _ssp(u)
    v = (
        jnp.dot(u, wf2_ref[...], preferred_element_type=jnp.float32)
        + bf2_ref[...]
    )
    v = _ssp(v)
    be, f = v.shape
    v3 = jnp.reshape(v, (be // 128, 128, f))
    e3 = lax.broadcast_in_dim(env_ref[0], (be // 128, 128, f), (0, 1))
    o_ref[:, 0:64] = jnp.reshape(v3 * e3, (be, f))


def _final_body(*args):
    o_ref = args[-1]
    x0_ref, w2_ref, b2_ref, w3_ref, b3_ref = args[-6:-1]
    p_refs = args[:-6]
    agg = p_refs[0][0] + p_refs[0][1]
    for p in p_refs[1:]:
        agg = agg + p[0] + p[1]
    u = (
        jnp.dot(agg, w2_ref[...], preferred_element_type=jnp.float32)
        + b2_ref[...]
    )
    u = _ssp(u)
    o_ref[...] = (
        jnp.dot(u, w3_ref[...], preferred_element_type=jnp.float32)
        + b3_ref[...]
        + x0_ref[...]
    )


_NC = 2
_NS = 16
_NW = _NC * _NS
_CH = 128


def _make_sc_scatter(N, F, off_chunks, cnt_chunks):
    T = (cnt_chunks + _NW - 1) // _NW
    rpt = (N // _NS) // 8 * 8
    rrem = N - rpt * _NS
    assert rrem % 8 == 0

    mesh = plsc.VectorSubcoreMesh(
        core_axis_name="c", subcore_axis_name="s", num_cores=_NC,
        num_subcores=_NS)

    @functools.partial(
        pl.kernel,
        out_type=jax.ShapeDtypeStruct((_NC * N, F), jnp.float32),
        mesh=mesh,
        scratch_types=[
            pltpu.VMEM((2, 2, _CH), jnp.int32),
            pltpu.VMEM((2, _CH, F), jnp.float32),
            pltpu.VMEM((2, _CH, 128), jnp.float32),
            pltpu.VMEM_SHARED((N, F), jnp.float32),
            pltpu.SemaphoreType.DMA((2,)),
            pltpu.SemaphoreType.DMA((2,)),
            pltpu.SemaphoreType.DMA((2,)),
        ],
        compiler_params=pltpu.CompilerParams(use_tc_tiling_on_sc=False),
    )
    def sc_scatter(ei_hbm, h_hbm, w_hbm, zero_hbm, out_hbm,
                   eib, hrows, wrows, acc, sidx, sw, sg):
        cid = lax.axis_index("c")
        sid = lax.axis_index("s")
        wid = sid * _NC + cid

        row0 = sid * rpt
        pltpu.sync_copy(zero_hbm.at[pl.ds(row0, rpt)],
                        acc.at[pl.ds(row0, rpt)])
        if rrem:
            @pl.when(sid == 0)
            def _():
                pltpu.sync_copy(zero_hbm.at[pl.ds(_NS * rpt, rrem)],
                                acc.at[pl.ds(_NS * rpt, rrem)])
        plsc.subcore_barrier()

        def slot_base(j):
            g = wid * T + j
            gc = jnp.minimum(g, cnt_chunks - 1)
            return (off_chunks + gc) * _CH, gc * _CH, g < cnt_chunks

        def start_pre(j, b):
            ebase, wbase, _ = slot_base(j)
            pltpu.async_copy(ei_hbm.at[:, pl.ds(ebase, _CH)], eib.at[b],
                             sidx.at[b])
            pltpu.async_copy(w_hbm.at[pl.ds(wbase, _CH)], wrows.at[b],
                             sw.at[b])

        def wait_pre(b):
            base0 = 0
            pltpu.make_async_copy(ei_hbm.at[:, pl.ds(base0, _CH)],
                                  eib.at[b], sidx.at[b]).wait()
            pltpu.make_async_copy(w_hbm.at[pl.ds(base0, _CH)],
                                  wrows.at[b], sw.at[b]).wait()

        def start_gather(b):
            pltpu.async_copy(h_hbm.at[eib.at[b, 1]], hrows.at[b], sg.at[b])

        def wait_gather(b):
            pltpu.make_async_copy(h_hbm.at[eib.at[b, 1]], hrows.at[b],
                                  sg.at[b]).wait()

        start_pre(0, 0)
        wait_pre(0)
        start_gather(0)
        start_pre(1, 1)

        for j in range(T):
            b = j % 2
            ob = 1 - b
            if j + 1 < T:
                wait_pre(ob)
                start_gather(ob)
            wait_gather(b)

            def mul_row(r, c2):
                for rr in range(2):
                    for k in range(F // 16):
                        sl = pl.ds(k * 16, 16)
                        hrows[b, 2 * r + rr, sl] = (
                            hrows[b, 2 * r + rr, sl]
                            * wrows[b, 2 * r + rr, sl])
                return c2

            lax.fori_loop(0, _CH // 2, mul_row, 0)
            _, _, live = slot_base(j)

            @pl.when(live)
            def _():
                pltpu.sync_copy(hrows.at[b], acc.at[eib.at[b, 0]], add=True)

            if j + 2 < T:
                start_pre(j + 2, b)

        plsc.subcore_barrier()
        pltpu.sync_copy(
            acc.at[pl.ds(row0, rpt)],
            out_hbm.at[pl.ds(cid * N + row0, rpt)])
        if rrem:
            @pl.when(sid == 0)
            def _():
                pltpu.sync_copy(
                    acc.at[pl.ds(_NS * rpt, rrem)],
                    out_hbm.at[pl.ds(cid * N + _NS * rpt, rrem)])

    return sc_scatter


def kernel(edge_index, node_feature, rbf_tensor, dist, cutoff,
           W1, b1, Wf1, bf1, Wf2, bf2, W2, b2, W3, b3):
    N, F = node_feature.shape
    E = edge_index.shape[1]
    R = rbf_tensor.shape[1]

    h = pl.pallas_call(
        _h_body,
        out_shape=jax.ShapeDtypeStruct((N, F), jnp.float32),
    )(node_feature, W1.T, b1.reshape(1, F))

    scale = (jnp.float32(3.14159265)
             / jnp.asarray(cutoff, jnp.float32)).reshape(1, 1)
    env2d = pl.pallas_call(
        _env_body,
        in_specs=[
            pl.BlockSpec(memory_space=pltpu.SMEM),
            pl.BlockSpec(memory_space=pltpu.VMEM),
        ],
        out_shape=jax.ShapeDtypeStruct((E // 128, 128), jnp.float32),
    )(scale, dist.reshape(E // 128, 128))

    BE = 640
    K = 5
    assert E % (K * BE) == 0 and BE % 128 == 0
    env3d = env2d.reshape(E // BE, BE // 128, 128)
    rbft = rbf_tensor.T
    seg_blocks = E // BE // K

    def filter_seg(k):
        goff = k * seg_blocks
        return pl.pallas_call(
            _filter_body,
            grid=(seg_blocks,),
            in_specs=[
                pl.BlockSpec((R, BE), lambda i: (0, i + goff)),
                pl.BlockSpec((1, BE // 128, 128),
                             lambda i: (i + goff, 0, 0)),
                pl.BlockSpec((R, F), lambda i: (0, 0)),
                pl.BlockSpec((1, F), lambda i: (0, 0)),
                pl.BlockSpec((F, F), lambda i: (0, 0)),
                pl.BlockSpec((1, F), lambda i: (0, 0)),
            ],
            out_specs=pl.BlockSpec((BE, 128), lambda i: (i, 0)),
            out_shape=jax.ShapeDtypeStruct((E // K, 128), jnp.float32),
        )(rbft, env3d, Wf1.T, bf1.reshape(1, F), Wf2.T, bf2.reshape(1, F))

    zeros = jnp.zeros((N, F), jnp.float32)
    seg_chunks = E // _CH // K
    parts = []
    for k in range(K):
        wk = filter_seg(k)
        parts.append(
            _make_sc_scatter(N, F, k * seg_chunks, seg_chunks)(
                edge_index, h, wk, zeros))

    BN = 2000
    assert N % BN == 0
    out = pl.pallas_call(
        _final_body,
        grid=(N // BN,),
        in_specs=[pl.BlockSpec((2, BN, F), lambda i: (0, i, 0))] * K + [
            pl.BlockSpec((BN, F), lambda i: (i, 0)),
            pl.BlockSpec((F, F), lambda i: (0, 0)),
            pl.BlockSpec((1, F), lambda i: (0, 0)),
            pl.BlockSpec((F, F), lambda i: (0, 0)),
            pl.BlockSpec((1, F), lambda i: (0, 0)),
        ],
        out_specs=pl.BlockSpec((BN, F), lambda i: (i, 0)),
        out_shape=jax.ShapeDtypeStruct((N, F), jnp.float32),
    )(*[p.reshape(2, N, F) for p in parts], node_feature, W2.T,
      b2.reshape(1, F), W3.T, b3.reshape(1, F))
    return out


# speedup vs baseline: 2.7869x; 1.0839x over previous
import functools

import jax
import jax.numpy as jnp
from jax import lax
from jax.experimental import pallas as pl
from jax.experimental.pallas import tpu as pltpu
from jax.experimental.pallas import tpu_sc as plsc

_LOG2 = 0.6931471805599453


def _ssp(x):
    return jnp.log(jnp.exp(x) + 1.0) - _LOG2


def _h_body(x_ref, w_ref, b_ref, o_ref):
    o_ref[...] = (
        jnp.dot(x_ref[...], w_ref[...], preferred_element_type=jnp.float32)
        + b_ref[...]
    )


def _env_body(s_ref, d_ref, o_ref):
    o_ref[...] = 1.0 + jnp.cos(d_ref[...] * s_ref[0, 0])


def _filter_body(rbft_ref, env_ref, wf1_ref, bf1_ref, wf2_ref, bf2_ref,
                 o_ref):
    u = lax.dot_general(
        rbft_ref[...], wf1_ref[...], (((0,), (0,)), ((), ())),
        preferred_element_type=jnp.float32,
    ) + bf1_ref[...]
    u = _ssp(u)
    v = (
        jnp.dot(u, wf2_ref[...], preferred_element_type=jnp.float32)
        + bf2_ref[...]
    )
    v = _ssp(v)
    be, f = v.shape
    v3 = jnp.reshape(v, (be // 128, 128, f))
    e3 = lax.broadcast_in_dim(env_ref[0], (be // 128, 128, f), (0, 1))
    o_ref[:, 0:64] = jnp.reshape(v3 * e3, (be, f))


def _final_body(*args):
    o_ref = args[-1]
    x0_ref, w2_ref, b2_ref, w3_ref, b3_ref = args[-6:-1]
    p_refs = args[:-6]
    agg = p_refs[0][0] + p_refs[0][1]
    for p in p_refs[1:]:
        agg = agg + p[0] + p[1]
    u = (
        jnp.dot(agg, w2_ref[...], preferred_element_type=jnp.float32)
        + b2_ref[...]
    )
    u = _ssp(u)
    o_ref[...] = (
        jnp.dot(u, w3_ref[...], preferred_element_type=jnp.float32)
        + b3_ref[...]
        + x0_ref[...]
    )


_NC = 2
_NS = 16
_NW = _NC * _NS
_CH = 128


def _make_sc_scatter(N, F, off_chunks, cnt_chunks):
    T = (cnt_chunks + _NW - 1) // _NW
    rpt = (N // _NS) // 8 * 8
    rrem = N - rpt * _NS
    assert rrem % 8 == 0

    mesh = plsc.VectorSubcoreMesh(
        core_axis_name="c", subcore_axis_name="s", num_cores=_NC,
        num_subcores=_NS)

    @functools.partial(
        pl.kernel,
        out_type=jax.ShapeDtypeStruct((_NC * N, F), jnp.float32),
        mesh=mesh,
        scratch_types=[
            pltpu.VMEM((2, 2, _CH), jnp.int32),
            pltpu.VMEM((2, _CH, F), jnp.float32),
            pltpu.VMEM((2, _CH, 128), jnp.float32),
            pltpu.VMEM_SHARED((N, F), jnp.float32),
            pltpu.SemaphoreType.DMA((2,)),
            pltpu.SemaphoreType.DMA((2,)),
            pltpu.SemaphoreType.DMA((2,)),
        ],
        compiler_params=pltpu.CompilerParams(use_tc_tiling_on_sc=False),
    )
    def sc_scatter(ei_hbm, h_hbm, w_hbm, zero_hbm, out_hbm,
                   eib, hrows, wrows, acc, sidx, sw, sg):
        cid = lax.axis_index("c")
        sid = lax.axis_index("s")
        wid = sid * _NC + cid

        row0 = sid * rpt
        pltpu.sync_copy(zero_hbm.at[pl.ds(row0, rpt)],
                        acc.at[pl.ds(row0, rpt)])
        if rrem:
            @pl.when(sid == 0)
            def _():
                pltpu.sync_copy(zero_hbm.at[pl.ds(_NS * rpt, rrem)],
                                acc.at[pl.ds(_NS * rpt, rrem)])
        plsc.subcore_barrier()

        def slot_base(j):
            g = wid * T + j
            gc = jnp.minimum(g, cnt_chunks - 1)
            return (off_chunks + gc) * _CH, gc * _CH, g < cnt_chunks

        def start_pre(j, b):
            ebase, wbase, _ = slot_base(j)
            pltpu.async_copy(ei_hbm.at[:, pl.ds(ebase, _CH)], eib.at[b],
                             sidx.at[b])
            pltpu.async_copy(w_hbm.at[pl.ds(wbase, _CH)], wrows.at[b],
                             sw.at[b])

        def wait_pre(b):
            base0 = 0
            pltpu.make_async_copy(ei_hbm.at[:, pl.ds(base0, _CH)],
                                  eib.at[b], sidx.at[b]).wait()
            pltpu.make_async_copy(w_hbm.at[pl.ds(base0, _CH)],
                                  wrows.at[b], sw.at[b]).wait()

        def start_gather(b):
            pltpu.async_copy(h_hbm.at[eib.at[b, 1]], hrows.at[b], sg.at[b])

        def wait_gather(b):
            pltpu.make_async_copy(h_hbm.at[eib.at[b, 1]], hrows.at[b],
                                  sg.at[b]).wait()

        start_pre(0, 0)
        wait_pre(0)
        start_gather(0)
        start_pre(1, 1)

        for j in range(T):
            b = j % 2
            ob = 1 - b
            if j + 1 < T:
                wait_pre(ob)
                start_gather(ob)
            wait_gather(b)

            def mul_row(r, c2):
                for rr in range(2):
                    for k in range(F // 16):
                        sl = pl.ds(k * 16, 16)
                        hrows[b, 2 * r + rr, sl] = (
                            hrows[b, 2 * r + rr, sl]
                            * wrows[b, 2 * r + rr, sl])
                return c2

            lax.fori_loop(0, _CH // 2, mul_row, 0)
            _, _, live = slot_base(j)

            @pl.when(live)
            def _():
                pltpu.sync_copy(hrows.at[b], acc.at[eib.at[b, 0]], add=True)

            if j + 2 < T:
                start_pre(j + 2, b)

        plsc.subcore_barrier()
        pltpu.sync_copy(
            acc.at[pl.ds(row0, rpt)],
            out_hbm.at[pl.ds(cid * N + row0, rpt)])
        if rrem:
            @pl.when(sid == 0)
            def _():
                pltpu.sync_copy(
                    acc.at[pl.ds(_NS * rpt, rrem)],
                    out_hbm.at[pl.ds(cid * N + _NS * rpt, rrem)])

    return sc_scatter


def kernel(edge_index, node_feature, rbf_tensor, dist, cutoff,
           W1, b1, Wf1, bf1, Wf2, bf2, W2, b2, W3, b3):
    N, F = node_feature.shape
    E = edge_index.shape[1]
    R = rbf_tensor.shape[1]

    h = pl.pallas_call(
        _h_body,
        out_shape=jax.ShapeDtypeStruct((N, F), jnp.float32),
    )(node_feature, W1.T, b1.reshape(1, F))

    scale = (jnp.float32(3.14159265)
             / jnp.asarray(cutoff, jnp.float32)).reshape(1, 1)
    env2d = pl.pallas_call(
        _env_body,
        in_specs=[
            pl.BlockSpec(memory_space=pltpu.SMEM),
            pl.BlockSpec(memory_space=pltpu.VMEM),
        ],
        out_shape=jax.ShapeDtypeStruct((E // 128, 128), jnp.float32),
    )(scale, dist.reshape(E // 128, 128))

    BE = 1280
    K = 5
    assert E % (K * BE) == 0 and BE % 128 == 0
    env3d = env2d.reshape(E // BE, BE // 128, 128)
    rbft = rbf_tensor.T
    seg_blocks = E // BE // K

    def filter_seg(k):
        goff = k * seg_blocks
        return pl.pallas_call(
            _filter_body,
            grid=(seg_blocks,),
            in_specs=[
                pl.BlockSpec((R, BE), lambda i: (0, i + goff)),
                pl.BlockSpec((1, BE // 128, 128),
                             lambda i: (i + goff, 0, 0)),
                pl.BlockSpec((R, F), lambda i: (0, 0)),
                pl.BlockSpec((1, F), lambda i: (0, 0)),
                pl.BlockSpec((F, F), lambda i: (0, 0)),
                pl.BlockSpec((1, F), lambda i: (0, 0)),
            ],
            out_specs=pl.BlockSpec((BE, 128), lambda i: (i, 0)),
            out_shape=jax.ShapeDtypeStruct((E // K, 128), jnp.float32),
        )(rbft, env3d, Wf1.T, bf1.reshape(1, F), Wf2.T, bf2.reshape(1, F))

    zeros = jnp.zeros((N, F), jnp.float32)
    seg_chunks = E // _CH // K
    parts = []
    for k in range(K):
        wk = filter_seg(k)
        parts.append(
            _make_sc_scatter(N, F, k * seg_chunks, seg_chunks)(
                edge_index, h, wk, zeros))

    BN = 2000
    assert N % BN == 0
    out = pl.pallas_call(
        _final_body,
        grid=(N // BN,),
        in_specs=[pl.BlockSpec((2, BN, F), lambda i: (0, i, 0))] * K + [
            pl.BlockSpec((BN, F), lambda i: (i, 0)),
            pl.BlockSpec((F, F), lambda i: (0, 0)),
            pl.BlockSpec((1, F), lambda i: (0, 0)),
            pl.BlockSpec((F, F), lambda i: (0, 0)),
            pl.BlockSpec((1, F), lambda i: (0, 0)),
        ],
        out_specs=pl.BlockSpec((BN, F), lambda i: (i, 0)),
        out_shape=jax.ShapeDtypeStruct((N, F), jnp.float32),
    )(*[p.reshape(2, N, F) for p in parts], node_feature, W2.T,
      b2.reshape(1, F), W3.T, b3.reshape(1, F))
    return out


# speedup vs baseline: 2.8333x; 1.0166x over previous
import functools

import jax
import jax.numpy as jnp
from jax import lax
from jax.experimental import pallas as pl
from jax.experimental.pallas import tpu as pltpu
from jax.experimental.pallas import tpu_sc as plsc

_LOG2 = 0.6931471805599453


def _ssp(x):
    return jnp.log(jnp.exp(x) + 1.0) - _LOG2


def _h_body(x_ref, w_ref, b_ref, o_ref):
    o_ref[...] = (
        jnp.dot(x_ref[...], w_ref[...], preferred_element_type=jnp.float32)
        + b_ref[...]
    )


def _env_body(s_ref, d_ref, o_ref):
    o_ref[...] = 1.0 + jnp.cos(d_ref[...] * s_ref[0, 0])


def _filter_body(rbft_ref, env_ref, wf1_ref, bf1_ref, wf2_ref, bf2_ref,
                 o_ref):
    u = lax.dot_general(
        rbft_ref[...], wf1_ref[...], (((0,), (0,)), ((), ())),
        preferred_element_type=jnp.float32,
    ) + bf1_ref[...]
    u = _ssp(u)
    v = (
        jnp.dot(u, wf2_ref[...], preferred_element_type=jnp.float32)
        + bf2_ref[...]
    )
    v = _ssp(v)
    be, f = v.shape
    v3 = jnp.reshape(v, (be // 128, 128, f))
    e3 = lax.broadcast_in_dim(env_ref[0], (be // 128, 128, f), (0, 1))
    o_ref[:, 0:64] = jnp.reshape(v3 * e3, (be, f))


def _final_body(*args):
    o_ref = args[-1]
    x0_ref, w2_ref, b2_ref, w3_ref, b3_ref = args[-6:-1]
    p_refs = args[:-6]
    agg = p_refs[0][0] + p_refs[0][1]
    for p in p_refs[1:]:
        agg = agg + p[0] + p[1]
    u = (
        jnp.dot(agg, w2_ref[...], preferred_element_type=jnp.float32)
        + b2_ref[...]
    )
    u = _ssp(u)
    o_ref[...] = (
        jnp.dot(u, w3_ref[...], preferred_element_type=jnp.float32)
        + b3_ref[...]
        + x0_ref[...]
    )


_NC = 2
_NS = 16
_NW = _NC * _NS
_CH = 128


def _make_sc_scatter(N, F, off_chunks, cnt_chunks):
    T = (cnt_chunks + _NW - 1) // _NW
    rpt = (N // _NS) // 8 * 8
    rrem = N - rpt * _NS
    assert rrem % 8 == 0

    mesh = plsc.VectorSubcoreMesh(
        core_axis_name="c", subcore_axis_name="s", num_cores=_NC,
        num_subcores=_NS)

    @functools.partial(
        pl.kernel,
        out_type=jax.ShapeDtypeStruct((_NC * N, F), jnp.float32),
        mesh=mesh,
        scratch_types=[
            pltpu.VMEM((2, 2, _CH), jnp.int32),
            pltpu.VMEM((2, _CH, F), jnp.float32),
            pltpu.VMEM((2, _CH, 128), jnp.float32),
            pltpu.VMEM_SHARED((N, F), jnp.float32),
            pltpu.SemaphoreType.DMA((2,)),
            pltpu.SemaphoreType.DMA((2,)),
            pltpu.SemaphoreType.DMA((2,)),
        ],
        compiler_params=pltpu.CompilerParams(use_tc_tiling_on_sc=False),
    )
    def sc_scatter(ei_hbm, h_hbm, w_hbm, zero_hbm, out_hbm,
                   eib, hrows, wrows, acc, sidx, sw, sg):
        cid = lax.axis_index("c")
        sid = lax.axis_index("s")
        wid = sid * _NC + cid

        row0 = sid * rpt
        pltpu.sync_copy(zero_hbm.at[pl.ds(row0, rpt)],
                        acc.at[pl.ds(row0, rpt)])
        if rrem:
            @pl.when(sid == 0)
            def _():
                pltpu.sync_copy(zero_hbm.at[pl.ds(_NS * rpt, rrem)],
                                acc.at[pl.ds(_NS * rpt, rrem)])
        plsc.subcore_barrier()

        def slot_base(j):
            g = wid * T + j
            gc = jnp.minimum(g, cnt_chunks - 1)
            return (off_chunks + gc) * _CH, gc * _CH, g < cnt_chunks

        def start_pre(j, b):
            ebase, wbase, _ = slot_base(j)
            pltpu.async_copy(ei_hbm.at[:, pl.ds(ebase, _CH)], eib.at[b],
                             sidx.at[b])
            pltpu.async_copy(w_hbm.at[pl.ds(wbase, _CH)], wrows.at[b],
                             sw.at[b])

        def wait_pre(b):
            base0 = 0
            pltpu.make_async_copy(ei_hbm.at[:, pl.ds(base0, _CH)],
                                  eib.at[b], sidx.at[b]).wait()
            pltpu.make_async_copy(w_hbm.at[pl.ds(base0, _CH)],
                                  wrows.at[b], sw.at[b]).wait()

        def start_gather(b):
            pltpu.async_copy(h_hbm.at[eib.at[b, 1]], hrows.at[b], sg.at[b])

        def wait_gather(b):
            pltpu.make_async_copy(h_hbm.at[eib.at[b, 1]], hrows.at[b],
                                  sg.at[b]).wait()

        start_pre(0, 0)
        wait_pre(0)
        start_gather(0)
        start_pre(1, 1)

        for j in range(T):
            b = j % 2
            ob = 1 - b
            if j + 1 < T:
                wait_pre(ob)
                start_gather(ob)
            wait_gather(b)

            def mul_row(r, c2):
                for rr in range(2):
                    for k in range(F // 16):
                        sl = pl.ds(k * 16, 16)
                        hrows[b, 2 * r + rr, sl] = (
                            hrows[b, 2 * r + rr, sl]
                            * wrows[b, 2 * r + rr, sl])
                return c2

            lax.fori_loop(0, _CH // 2, mul_row, 0)
            _, _, live = slot_base(j)

            @pl.when(live)
            def _():
                pltpu.sync_copy(hrows.at[b], acc.at[eib.at[b, 0]], add=True)

            if j + 2 < T:
                start_pre(j + 2, b)

        plsc.subcore_barrier()
        pltpu.sync_copy(
            acc.at[pl.ds(row0, rpt)],
            out_hbm.at[pl.ds(cid * N + row0, rpt)])
        if rrem:
            @pl.when(sid == 0)
            def _():
                pltpu.sync_copy(
                    acc.at[pl.ds(_NS * rpt, rrem)],
                    out_hbm.at[pl.ds(cid * N + _NS * rpt, rrem)])

    return sc_scatter


def kernel(edge_index, node_feature, rbf_tensor, dist, cutoff,
           W1, b1, Wf1, bf1, Wf2, bf2, W2, b2, W3, b3):
    N, F = node_feature.shape
    E = edge_index.shape[1]
    R = rbf_tensor.shape[1]

    h = pl.pallas_call(
        _h_body,
        out_shape=jax.ShapeDtypeStruct((N, F), jnp.float32),
    )(node_feature, W1.T, b1.reshape(1, F))

    scale = (jnp.float32(3.14159265)
             / jnp.asarray(cutoff, jnp.float32)).reshape(1, 1)
    env2d = pl.pallas_call(
        _env_body,
        in_specs=[
            pl.BlockSpec(memory_space=pltpu.SMEM),
            pl.BlockSpec(memory_space=pltpu.VMEM),
        ],
        out_shape=jax.ShapeDtypeStruct((E // 128, 128), jnp.float32),
    )(scale, dist.reshape(E // 128, 128))

    BE = 3200
    K = 5
    assert E % (K * BE) == 0 and BE % 128 == 0
    env3d = env2d.reshape(E // BE, BE // 128, 128)
    rbft = rbf_tensor.T
    seg_blocks = E // BE // K

    def filter_seg(k):
        goff = k * seg_blocks
        return pl.pallas_call(
            _filter_body,
            grid=(seg_blocks,),
            in_specs=[
                pl.BlockSpec((R, BE), lambda i: (0, i + goff)),
                pl.BlockSpec((1, BE // 128, 128),
                             lambda i: (i + goff, 0, 0)),
                pl.BlockSpec((R, F), lambda i: (0, 0)),
                pl.BlockSpec((1, F), lambda i: (0, 0)),
                pl.BlockSpec((F, F), lambda i: (0, 0)),
                pl.BlockSpec((1, F), lambda i: (0, 0)),
            ],
            out_specs=pl.BlockSpec((BE, 128), lambda i: (i, 0)),
            out_shape=jax.ShapeDtypeStruct((E // K, 128), jnp.float32),
        )(rbft, env3d, Wf1.T, bf1.reshape(1, F), Wf2.T, bf2.reshape(1, F))

    zeros = jnp.zeros((N, F), jnp.float32)
    seg_chunks = E // _CH // K
    parts = []
    for k in range(K):
        wk = filter_seg(k)
        parts.append(
            _make_sc_scatter(N, F, k * seg_chunks, seg_chunks)(
                edge_index, h, wk, zeros))

    BN = 2000
    assert N % BN == 0
    out = pl.pallas_call(
        _final_body,
        grid=(N // BN,),
        in_specs=[pl.BlockSpec((2, BN, F), lambda i: (0, i, 0))] * K + [
            pl.BlockSpec((BN, F), lambda i: (i, 0)),
            pl.BlockSpec((F, F), lambda i: (0, 0)),
            pl.BlockSpec((1, F), lambda i: (0, 0)),
            pl.BlockSpec((F, F), lambda i: (0, 0)),
            pl.BlockSpec((1, F), lambda i: (0, 0)),
        ],
        out_specs=pl.BlockSpec((BN, F), lambda i: (i, 0)),
        out_shape=jax.ShapeDtypeStruct((N, F), jnp.float32),
    )(*[p.reshape(2, N, F) for p in parts], node_feature, W2.T,
      b2.reshape(1, F), W3.T, b3.reshape(1, F))
    return out
